# baseline (speedup 1.0000x reference)
import functools

import jax
import jax.numpy as jnp
from jax import lax
from jax.experimental import pallas as pl
from jax.experimental.pallas import tpu as pltpu
from jax.experimental.pallas import tpu_sc as plsc

B = 4096
D = 1024
H = D // 2
E = 8
K = 2
TM = 512
TM2 = 256
LOG_TM2 = 8
PMAX = K * B + E * TM2
NT2 = PMAX // TM2
NTE = 48
NW = 32
TPW = B // NW
CT = 16
NCH = TPW // CT


def _rne_bf16_bits(v):
    u = jax.lax.bitcast_convert_type(v, jnp.uint32)
    return (u + jnp.uint32(0x7FFF) + ((u >> 16) & jnp.uint32(1))) >> 16


def _pack_bf16(v):
    r = _rne_bf16_bits(v)
    pk = r[:, :H] | (r[:, H:] << 16)
    return jax.lax.bitcast_convert_type(pk, jnp.int32)


def _unpack_bf16(pk):
    lo = jax.lax.bitcast_convert_type(pk << 16, jnp.float32)
    hi = jax.lax.bitcast_convert_type(pk & jnp.int32(-65536), jnp.float32)
    return jnp.concatenate([lo, hi], axis=1)


def _routing_body(x_ref, wg_ref, bg_ref, xpk_ref, meta_ref, wm_ref,
                  cnt_ref, imp_ref, ll_ref, il_ref):
    i = pl.program_id(0)

    @pl.when(i == 0)
    def _():
        cnt_ref[...] = jnp.zeros((1, 16), jnp.float32)
        imp_ref[...] = jnp.zeros((1, E), jnp.float32)

    cb = cnt_ref[...][:, :E]
    x = x_ref[...]
    xpk_ref[...] = _pack_bf16(x)
    s = jax.lax.dot_general(
        x, wg_ref[...], (((1,), (1,)), ((), ())),
        preferred_element_type=jnp.float32) + bg_ref[...]
    ids = jax.lax.broadcasted_iota(jnp.int32, (TM, E), 1)
    m1 = jnp.max(s, axis=1, keepdims=True)
    a1v = jnp.min(jnp.where(s == m1, ids, E), axis=1, keepdims=True)
    s2 = jnp.where(ids == a1v, -jnp.inf, s)
    m2 = jnp.max(s2, axis=1, keepdims=True)
    a2v = jnp.min(jnp.where(s2 == m2, ids, E), axis=1, keepdims=True)
    e21 = jnp.exp(m2 - m1)
    w0 = 1.0 / (1.0 + e21)
    w1 = e21 / (1.0 + e21)
    wbits = (_rne_bf16_bits(w0) | (_rne_bf16_bits(w1) << 16))
    wm_ref[...] = jax.lax.bitcast_convert_type(wbits, jnp.int32)
    is1 = (ids == a1v).astype(jnp.float32)
    is2 = (ids == a2v).astype(jnp.float32)
    m = is1 + is2
    c = m
    sh = 1
    while sh < TM:
        c = c + jnp.concatenate(
            [jnp.zeros((sh, E), jnp.float32), c[:TM - sh]], axis=0)
        sh *= 2
    cexc = c - m
    r0 = jnp.sum(is1 * (cexc + cb), axis=1, keepdims=True).astype(jnp.int32)
    r1 = jnp.sum(is2 * (cexc + is1 + cb), axis=1,
                 keepdims=True).astype(jnp.int32)
    meta_ref[...] = a1v | (r0 << 3) | (a2v << 16) | (r1 << 19)
    cpart = jnp.sum(m, axis=0, keepdims=True)
    cnt_ref[...] += jnp.concatenate(
        [cpart, jnp.zeros((1, 16 - E), jnp.float32)], axis=1)
    ex = jnp.exp(s - m1)
    sm = ex / jnp.sum(ex, axis=1, keepdims=True)
    imp_ref[...] += jnp.sum(sm, axis=0, keepdims=True)

    @pl.when(i == pl.num_programs(0) - 1)
    def _():
        cfin = cnt_ref[...][:, :E]
        cm = jnp.sum(cfin) / E
        cvar = jnp.sum((cfin - cm) ** 2) / (E - 1)
        ll_ref[...] = cvar.reshape(1, 1) / (E * (B / E))
        im = imp_ref[...]
        imm = jnp.sum(im) / E
        ivar = jnp.sum((im - imm) ** 2) / (E - 1)
        il_ref[...] = ivar.reshape(1, 1) / (imm + 1e-8)


def _pos_body(meta_ref, cnt_ref, p0_ref, p1_ref, te_ref):
    m = meta_ref[...]
    a0 = m & 7
    r0 = (m >> 3) & 0x1FFF
    a1 = (m >> 16) & 7
    r1 = lax.shift_right_logical(m, 19)
    c = cnt_ref[...][:, :E]
    pc = jnp.ceil(c / TM2) * TM2
    lt = (jax.lax.broadcasted_iota(jnp.int32, (E, E), 0) <
          jax.lax.broadcasted_iota(jnp.int32, (E, E), 1)).astype(jnp.float32)
    offs = jax.lax.dot_general(pc, lt, (((1,), (0,)), ((), ())),
                               preferred_element_type=jnp.float32)
    iot = jax.lax.broadcasted_iota(jnp.int32, (TM, E), 1)
    for a, r, p_ref in ((a0, r0, p0_ref), (a1, r1, p1_ref)):
        oh = (a == iot).astype(jnp.float32)
        osel = jnp.sum(oh * offs, axis=1, keepdims=True)
        p_ref[...] = osel.astype(jnp.int32) + r

    @pl.when(pl.program_id(0) == 0)
    def _():
        ends = offs + pc
        starts = (jax.lax.broadcasted_iota(jnp.int32, (NTE, 1), 0)
                  * TM2).astype(jnp.float32)
        cmp = (starts >= ends).astype(jnp.int32)
        te_ref[...] = jnp.minimum(jnp.sum(cmp, axis=1, keepdims=True), E - 1)


def _gmlp_body(te_ref, xs_ref, w1_ref, b1_ref, w2_ref, b2_ref, o_ref):
    xb = _unpack_bf16(xs_ref[...]).astype(jnp.bfloat16)
    h = jax.lax.dot_general(
        xb, w1_ref[0], (((1,), (1,)), ((), ())),
        preferred_element_type=jnp.float32) + b1_ref[0]
    hb = jnp.maximum(h, 0.0).astype(jnp.bfloat16)
    o = jax.lax.dot_general(
        hb, w2_ref[0], (((1,), (1,)), ((), ())),
        preferred_element_type=jnp.float32) + b2_ref[0]
    o_ref[...] = _pack_bf16(o)


def _offsets(cnt_v, offs_v):
    ci = cnt_v[...].astype(jnp.int32)
    pci = ((ci + (TM2 - 1)) >> LOG_TM2) << LOG_TM2
    incl = plsc.cumsum(pci)
    excl = incl - pci
    offs_v[...] = excl
    return excl, pci


def _slots(m, offs_v):
    a0 = m & 7
    r0 = (m >> 3) & 0x1FFF
    a1 = (m >> 16) & 7
    r1 = lax.shift_right_logical(m, 19)
    s0 = r0 + plsc.load_gather(offs_v, [a0])
    s1 = r1 + plsc.load_gather(offs_v, [a1])
    return s0, s1


def _dispatch_body(xpk_hbm, p0_hbm, p1_hbm, xs_hbm,
                   rows_v, i0_v, i1_v, seml, sem0, sem1):
    wid = lax.axis_index("s") * 2 + lax.axis_index("c")
    base = wid * TPW
    l0 = pltpu.async_copy(p0_hbm.at[pl.ds(base, TPW)], i0_v, seml)
    l1 = pltpu.async_copy(p1_hbm.at[pl.ds(base, TPW)], i1_v, seml)
    l2 = pltpu.async_copy(xpk_hbm.at[pl.ds(base, TPW)], rows_v, seml)
    l0.wait()
    l1.wait()
    l2.wait()
    c0 = pltpu.async_copy(rows_v, xs_hbm.at[i0_v], sem0)
    c1 = pltpu.async_copy(rows_v, xs_hbm.at[i1_v], sem1)
    c0.wait()
    c1.wait()


def _combine_body(os_hbm, meta_hbm, wm_hbm, cnt_hbm, out_hbm,
                  m_v, wm_v, cnt_v, offs_v,
                  ia0_v, ia1_v, ib0_v, ib1_v,
                  ra0_v, ra1_v, rb0_v, rb1_v, oc0_v, oc1_v,
                  seml, sema0, sema1, semb0, semb1, semo0, semo1):
    wid = lax.axis_index("s") * 2 + lax.axis_index("c")
    base = wid * TPW
    l0 = pltpu.async_copy(meta_hbm.at[pl.ds(base, TPW)], m_v, seml)
    l1 = pltpu.async_copy(wm_hbm.at[pl.ds(base, TPW)], wm_v, seml)
    pltpu.sync_copy(cnt_hbm, cnt_v)
    _offsets(cnt_v, offs_v)
    l0.wait()
    l1.wait()
    ia = (ia0_v, ia1_v)
    ib = (ib0_v, ib1_v)
    ra = (ra0_v, ra1_v)
    rb = (rb0_v, rb1_v)
    oc = (oc0_v, oc1_v)
    sa = (sema0, sema1)
    sb = (semb0, semb1)
    so = (semo0, semo1)
    gat = [None, None]
    odma = [None, None]

    def start(ci):
        nb = ci % 2
        m = m_v[pl.ds(ci * CT, CT)]
        s0, s1 = _slots(m, offs_v)
        ia[nb][...] = s0
        ib[nb][...] = s1
        gat[nb] = (pltpu.async_copy(os_hbm.at[ia[nb]], ra[nb], sa[nb]),
                   pltpu.async_copy(os_hbm.at[ib[nb]], rb[nb], sb[nb]))

    start(0)
    for ci in range(NCH):
        nb = ci % 2
        if ci + 1 < NCH:
            start(ci + 1)
        gat[nb][0].wait()
        gat[nb][1].wait()
        if odma[nb] is not None:
            odma[nb].wait()
        ra_v = ra[nb]
        rb_v = rb[nb]
        out_v = oc[nb]

        def tok_body(t, carry):
            wm = plsc.load_gather(wm_v, [jnp.full((16,), ci * CT + t,
                                                  jnp.int32)])
            g0 = plsc.bitcast(wm << 16, jnp.float32)
            g1 = plsc.bitcast(wm & -65536, jnp.float32)
            for dc in range(H // 16):
                off = dc * 16
                ai = ra_v[t, pl.ds(off, 16)]
                bi = rb_v[t, pl.ds(off, 16)]
                alo = plsc.bitcast(ai << 16, jnp.float32)
                ahi = plsc.bitcast(ai & -65536, jnp.float32)
                blo = plsc.bitcast(bi << 16, jnp.float32)
                bhi = plsc.bitcast(bi & -65536, jnp.float32)
                out_v[t, pl.ds(off, 16)] = g0 * alo + g1 * blo
                out_v[t, pl.ds(off + H, 16)] = g0 * ahi + g1 * bhi
            return carry

        lax.fori_loop(0, CT, tok_body, 0)
        odma[nb] = pltpu.async_copy(
            out_v, out_hbm.at[pl.ds(base + ci * CT, CT)], so[nb])
    for nb in range(2):
        if odma[nb] is not None:
            odma[nb].wait()


def kernel(x, Wg, bg, W1, b1, W2, b2):
    nt = B // TM
    f32 = jnp.float32
    xpk, meta, wm, cnt, imp, ll, il = pl.pallas_call(
        _routing_body,
        grid=(nt,),
        in_specs=[
            pl.BlockSpec((TM, D), lambda i: (i, 0)),
            pl.BlockSpec((E, D), lambda i: (0, 0)),
            pl.BlockSpec((1, E), lambda i: (0, 0)),
        ],
        out_specs=[
            pl.BlockSpec((TM, H), lambda i: (i, 0)),
            pl.BlockSpec((TM, 1), lambda i: (i, 0)),
            pl.BlockSpec((TM, 1), lambda i: (i, 0)),
            pl.BlockSpec((1, 16), lambda i: (0, 0)),
            pl.BlockSpec((1, E), lambda i: (0, 0)),
            pl.BlockSpec((1, 1), lambda i: (0, 0)),
            pl.BlockSpec((1, 1), lambda i: (0, 0)),
        ],
        out_shape=[
            jax.ShapeDtypeStruct((B, H), jnp.int32),
            jax.ShapeDtypeStruct((B, 1), jnp.int32),
            jax.ShapeDtypeStruct((B, 1), jnp.int32),
            jax.ShapeDtypeStruct((1, 16), f32),
            jax.ShapeDtypeStruct((1, E), f32),
            jax.ShapeDtypeStruct((1, 1), f32),
            jax.ShapeDtypeStruct((1, 1), f32),
        ],
    )(x, Wg, bg.reshape(1, E))

    metaf = meta.reshape(B)
    wmf = wm.reshape(B)
    cntf = cnt.reshape(16)
    p0, p1, te = pl.pallas_call(
        _pos_body,
        grid=(nt,),
        in_specs=[
            pl.BlockSpec((TM, 1), lambda i: (i, 0)),
            pl.BlockSpec((1, 16), lambda i: (0, 0)),
        ],
        out_specs=[
            pl.BlockSpec((TM, 1), lambda i: (i, 0)),
            pl.BlockSpec((TM, 1), lambda i: (i, 0)),
            pl.BlockSpec((NTE, 1), lambda i: (0, 0)),
        ],
        out_shape=[
            jax.ShapeDtypeStruct((B, 1), jnp.int32),
            jax.ShapeDtypeStruct((B, 1), jnp.int32),
            jax.ShapeDtypeStruct((NTE, 1), jnp.int32),
        ],
    )(meta, cnt)
    te = te.reshape(NTE)
    xs = _sc_dispatch(xpk, p0.reshape(B), p1.reshape(B))

    w1b = W1.astype(jnp.bfloat16)
    w2b = W2.astype(jnp.bfloat16)
    grid_spec = pltpu.PrefetchScalarGridSpec(
        num_scalar_prefetch=1,
        grid=(NT2,),
        in_specs=[
            pl.BlockSpec((TM2, H), lambda i, te_r: (i, 0)),
            pl.BlockSpec((1, D, D), lambda i, te_r: (te_r[i], 0, 0)),
            pl.BlockSpec((1, 1, D), lambda i, te_r: (te_r[i], 0, 0)),
            pl.BlockSpec((1, D, D), lambda i, te_r: (te_r[i], 0, 0)),
            pl.BlockSpec((1, 1, D), lambda i, te_r: (te_r[i], 0, 0)),
        ],
        out_specs=pl.BlockSpec((TM2, H), lambda i, te_r: (i, 0)),
    )
    os_rows = pl.pallas_call(
        _gmlp_body,
        grid_spec=grid_spec,
        out_shape=jax.ShapeDtypeStruct((PMAX, H), jnp.int32),
    )(te[:NT2], xs, w1b, b1.reshape(E, 1, D), w2b, b2.reshape(E, 1, D))

    out = _sc_combine(os_rows, metaf, wmf, cntf)

    return out, ll.reshape(()), il.reshape(())


def _sc_mesh():
    return plsc.VectorSubcoreMesh(core_axis_name="c", subcore_axis_name="s",
                                  num_cores=2, num_subcores=16)


def _sc_dispatch(xpk, p0f, p1f):
    dispatch = functools.partial(
        pl.kernel,
        out_type=jax.ShapeDtypeStruct((PMAX, H), jnp.int32),
        mesh=_sc_mesh(),
        scratch_types=[
            pltpu.VMEM((TPW, H), jnp.int32),
            pltpu.VMEM((TPW,), jnp.int32),
            pltpu.VMEM((TPW,), jnp.int32),
            pltpu.SemaphoreType.DMA,
            pltpu.SemaphoreType.DMA,
            pltpu.SemaphoreType.DMA,
        ],
    )(_dispatch_body)
    return dispatch(xpk, p0f, p1f)


def _sc_combine(os_rows, metaf, wmf, cntf):
    f32 = jnp.float32
    combine = functools.partial(
        pl.kernel,
        out_type=jax.ShapeDtypeStruct((B, D), f32),
        mesh=_sc_mesh(),
        compiler_params=pltpu.CompilerParams(needs_layout_passes=False),
        scratch_types=[
            pltpu.VMEM((TPW,), jnp.int32),
            pltpu.VMEM((TPW,), jnp.int32),
            pltpu.VMEM((16,), f32),
            pltpu.VMEM((16,), jnp.int32),
            pltpu.VMEM((CT,), jnp.int32),
            pltpu.VMEM((CT,), jnp.int32),
            pltpu.VMEM((CT,), jnp.int32),
            pltpu.VMEM((CT,), jnp.int32),
            pltpu.VMEM((CT, H), jnp.int32),
            pltpu.VMEM((CT, H), jnp.int32),
            pltpu.VMEM((CT, H), jnp.int32),
            pltpu.VMEM((CT, H), jnp.int32),
            pltpu.VMEM((CT, D), f32),
            pltpu.VMEM((CT, D), f32),
            pltpu.SemaphoreType.DMA,
            pltpu.SemaphoreType.DMA,
            pltpu.SemaphoreType.DMA,
            pltpu.SemaphoreType.DMA,
            pltpu.SemaphoreType.DMA,
            pltpu.SemaphoreType.DMA,
            pltpu.SemaphoreType.DMA,
        ],
    )(_combine_body)
    return combine(os_rows, metaf, wmf, cntf)


# speedup vs baseline: 1.0036x; 1.0036x over previous
import functools

import jax
import jax.numpy as jnp
from jax import lax
from jax.experimental import pallas as pl
from jax.experimental.pallas import tpu as pltpu
from jax.experimental.pallas import tpu_sc as plsc

B = 4096
D = 1024
H = D // 2
E = 8
K = 2
TM = 512
TM2 = 256
LOG_TM2 = 8
PMAX = K * B + E * TM2
NT2 = PMAX // TM2
NTE = 48
NW = 32
TPW = B // NW
CT = 16
NCH = TPW // CT


def _rne_bf16_bits(v):
    u = jax.lax.bitcast_convert_type(v, jnp.uint32)
    return (u + jnp.uint32(0x7FFF) + ((u >> 16) & jnp.uint32(1))) >> 16


def _pack_bf16(v):
    r = _rne_bf16_bits(v)
    pk = r[:, :H] | (r[:, H:] << 16)
    return jax.lax.bitcast_convert_type(pk, jnp.int32)


def _unpack_bf16(pk):
    lo = jax.lax.bitcast_convert_type(pk << 16, jnp.float32)
    hi = jax.lax.bitcast_convert_type(pk & jnp.int32(-65536), jnp.float32)
    return jnp.concatenate([lo, hi], axis=1)


def _routing_body(x_ref, wg_ref, bg_ref, xpk_ref, meta_ref, wm_ref,
                  cnt_ref, imp_ref, ll_ref, il_ref, p0_ref, p1_ref, te_ref,
                  mscr):
    i = pl.program_id(0)
    nt = B // TM

    @pl.when(i == 0)
    def _():
        cnt_ref[...] = jnp.zeros((1, 16), jnp.float32)
        imp_ref[...] = jnp.zeros((1, E), jnp.float32)

    @pl.when(i < nt)
    def _():
        cb = cnt_ref[...][:, :E]
        x = x_ref[...]
        xpk_ref[...] = _pack_bf16(x)
        s = jax.lax.dot_general(
            x, wg_ref[...], (((1,), (1,)), ((), ())),
            preferred_element_type=jnp.float32) + bg_ref[...]
        ids = jax.lax.broadcasted_iota(jnp.int32, (TM, E), 1)
        m1 = jnp.max(s, axis=1, keepdims=True)
        a1v = jnp.min(jnp.where(s == m1, ids, E), axis=1, keepdims=True)
        s2 = jnp.where(ids == a1v, -jnp.inf, s)
        m2 = jnp.max(s2, axis=1, keepdims=True)
        a2v = jnp.min(jnp.where(s2 == m2, ids, E), axis=1, keepdims=True)
        e21 = jnp.exp(m2 - m1)
        w0 = 1.0 / (1.0 + e21)
        w1 = e21 / (1.0 + e21)
        wbits = (_rne_bf16_bits(w0) | (_rne_bf16_bits(w1) << 16))
        wm_ref[...] = jax.lax.bitcast_convert_type(wbits, jnp.int32)
        is1 = (ids == a1v).astype(jnp.float32)
        is2 = (ids == a2v).astype(jnp.float32)
        m = is1 + is2
        c = m
        sh = 1
        while sh < TM:
            c = c + jnp.concatenate(
                [jnp.zeros((sh, E), jnp.float32), c[:TM - sh]], axis=0)
            sh *= 2
        cexc = c - m
        r0 = jnp.sum(is1 * (cexc + cb), axis=1,
                     keepdims=True).astype(jnp.int32)
        r1 = jnp.sum(is2 * (cexc + is1 + cb), axis=1,
                     keepdims=True).astype(jnp.int32)
        mv = a1v | (r0 << 3) | (a2v << 16) | (r1 << 19)
        meta_ref[...] = mv
        mscr[pl.ds(i * TM, TM), :] = mv
        cpart = jnp.sum(m, axis=0, keepdims=True)
        cnt_ref[...] += jnp.concatenate(
            [cpart, jnp.zeros((1, 16 - E), jnp.float32)], axis=1)
        ex = jnp.exp(s - m1)
        sm = ex / jnp.sum(ex, axis=1, keepdims=True)
        imp_ref[...] += jnp.sum(sm, axis=0, keepdims=True)

    @pl.when(i == nt - 1)
    def _():
        cfin = cnt_ref[...][:, :E]
        cm = jnp.sum(cfin) / E
        cvar = jnp.sum((cfin - cm) ** 2) / (E - 1)
        ll_ref[...] = cvar.reshape(1, 1) / (E * (B / E))
        im = imp_ref[...]
        imm = jnp.sum(im) / E
        ivar = jnp.sum((im - imm) ** 2) / (E - 1)
        il_ref[...] = ivar.reshape(1, 1) / (imm + 1e-8)

    @pl.when(i >= nt)
    def _():
        c = cnt_ref[...][:, :E]
        pc = jnp.ceil(c / TM2) * TM2
        lt = (jax.lax.broadcasted_iota(jnp.int32, (E, E), 0) <
              jax.lax.broadcasted_iota(jnp.int32, (E, E), 1)
              ).astype(jnp.float32)
        offs = jax.lax.dot_general(pc, lt, (((1,), (0,)), ((), ())),
                                   preferred_element_type=jnp.float32)
        mv = mscr[pl.ds((i - nt) * TM, TM), :]
        a0 = mv & 7
        r0 = (mv >> 3) & 0x1FFF
        a1 = (mv >> 16) & 7
        r1 = lax.shift_right_logical(mv, 19)
        iot = jax.lax.broadcasted_iota(jnp.int32, (TM, E), 1)
        for a, r, p_ref in ((a0, r0, p0_ref), (a1, r1, p1_ref)):
            oh = (a == iot).astype(jnp.float32)
            osel = jnp.sum(oh * offs, axis=1, keepdims=True)
            p_ref[...] = osel.astype(jnp.int32) + r

        @pl.when(i == nt)
        def _():
            ends = offs + pc
            starts = (jax.lax.broadcasted_iota(jnp.int32, (NTE, 1), 0)
                      * TM2).astype(jnp.float32)
            cmp = (starts >= ends).astype(jnp.int32)
            te_ref[...] = jnp.minimum(jnp.sum(cmp, axis=1, keepdims=True),
                                      E - 1)


def _pos_body(meta_ref, cnt_ref, p0_ref, p1_ref, te_ref):
    m = meta_ref[...]
    a0 = m & 7
    r0 = (m >> 3) & 0x1FFF
    a1 = (m >> 16) & 7
    r1 = lax.shift_right_logical(m, 19)
    c = cnt_ref[...][:, :E]
    pc = jnp.ceil(c / TM2) * TM2
    lt = (jax.lax.broadcasted_iota(jnp.int32, (E, E), 0) <
          jax.lax.broadcasted_iota(jnp.int32, (E, E), 1)).astype(jnp.float32)
    offs = jax.lax.dot_general(pc, lt, (((1,), (0,)), ((), ())),
                               preferred_element_type=jnp.float32)
    iot = jax.lax.broadcasted_iota(jnp.int32, (TM, E), 1)
    for a, r, p_ref in ((a0, r0, p0_ref), (a1, r1, p1_ref)):
        oh = (a == iot).astype(jnp.float32)
        osel = jnp.sum(oh * offs, axis=1, keepdims=True)
        p_ref[...] = osel.astype(jnp.int32) + r

    @pl.when(pl.program_id(0) == 0)
    def _():
        ends = offs + pc
        starts = (jax.lax.broadcasted_iota(jnp.int32, (NTE, 1), 0)
                  * TM2).astype(jnp.float32)
        cmp = (starts >= ends).astype(jnp.int32)
        te_ref[...] = jnp.minimum(jnp.sum(cmp, axis=1, keepdims=True), E - 1)


def _gmlp_body(te_ref, xs_ref, w1_ref, b1_ref, w2_ref, b2_ref, o_ref):
    xb = _unpack_bf16(xs_ref[...]).astype(jnp.bfloat16)
    h = jax.lax.dot_general(
        xb, w1_ref[0], (((1,), (1,)), ((), ())),
        preferred_element_type=jnp.float32) + b1_ref[0]
    hb = jnp.maximum(h, 0.0).astype(jnp.bfloat16)
    o = jax.lax.dot_general(
        hb, w2_ref[0], (((1,), (1,)), ((), ())),
        preferred_element_type=jnp.float32) + b2_ref[0]
    o_ref[...] = _pack_bf16(o)


def _offsets(cnt_v, offs_v):
    ci = cnt_v[...].astype(jnp.int32)
    pci = ((ci + (TM2 - 1)) >> LOG_TM2) << LOG_TM2
    incl = plsc.cumsum(pci)
    excl = incl - pci
    offs_v[...] = excl
    return excl, pci


def _slots(m, offs_v):
    a0 = m & 7
    r0 = (m >> 3) & 0x1FFF
    a1 = (m >> 16) & 7
    r1 = lax.shift_right_logical(m, 19)
    s0 = r0 + plsc.load_gather(offs_v, [a0])
    s1 = r1 + plsc.load_gather(offs_v, [a1])
    return s0, s1


def _dispatch_body(xpk_hbm, p0_hbm, p1_hbm, xs_hbm,
                   rows_v, i0_v, i1_v, seml, sem0, sem1):
    wid = lax.axis_index("s") * 2 + lax.axis_index("c")
    base = wid * TPW
    l0 = pltpu.async_copy(p0_hbm.at[pl.ds(base, TPW)], i0_v, seml)
    l1 = pltpu.async_copy(p1_hbm.at[pl.ds(base, TPW)], i1_v, seml)
    l2 = pltpu.async_copy(xpk_hbm.at[pl.ds(base, TPW)], rows_v, seml)
    l0.wait()
    l1.wait()
    l2.wait()
    c0 = pltpu.async_copy(rows_v, xs_hbm.at[i0_v], sem0)
    c1 = pltpu.async_copy(rows_v, xs_hbm.at[i1_v], sem1)
    c0.wait()
    c1.wait()


def _combine_body(os_hbm, meta_hbm, wm_hbm, cnt_hbm, out_hbm,
                  m_v, wm_v, cnt_v, offs_v,
                  ia0_v, ia1_v, ib0_v, ib1_v,
                  ra0_v, ra1_v, rb0_v, rb1_v, oc0_v, oc1_v,
                  seml, sema0, sema1, semb0, semb1, semo0, semo1):
    wid = lax.axis_index("s") * 2 + lax.axis_index("c")
    base = wid * TPW
    l0 = pltpu.async_copy(meta_hbm.at[pl.ds(base, TPW)], m_v, seml)
    l1 = pltpu.async_copy(wm_hbm.at[pl.ds(base, TPW)], wm_v, seml)
    pltpu.sync_copy(cnt_hbm, cnt_v)
    _offsets(cnt_v, offs_v)
    l0.wait()
    l1.wait()
    ia = (ia0_v, ia1_v)
    ib = (ib0_v, ib1_v)
    ra = (ra0_v, ra1_v)
    rb = (rb0_v, rb1_v)
    oc = (oc0_v, oc1_v)
    sa = (sema0, sema1)
    sb = (semb0, semb1)
    so = (semo0, semo1)
    gat = [None, None]
    odma = [None, None]

    def start(ci):
        nb = ci % 2
        m = m_v[pl.ds(ci * CT, CT)]
        s0, s1 = _slots(m, offs_v)
        ia[nb][...] = s0
        ib[nb][...] = s1
        gat[nb] = (pltpu.async_copy(os_hbm.at[ia[nb]], ra[nb], sa[nb]),
                   pltpu.async_copy(os_hbm.at[ib[nb]], rb[nb], sb[nb]))

    start(0)
    for ci in range(NCH):
        nb = ci % 2
        if ci + 1 < NCH:
            start(ci + 1)
        gat[nb][0].wait()
        gat[nb][1].wait()
        if odma[nb] is not None:
            odma[nb].wait()
        ra_v = ra[nb]
        rb_v = rb[nb]
        out_v = oc[nb]

        def tok_body(t, carry):
            wm = plsc.load_gather(wm_v, [jnp.full((16,), ci * CT + t,
                                                  jnp.int32)])
            g0 = plsc.bitcast(wm << 16, jnp.float32)
            g1 = plsc.bitcast(wm & -65536, jnp.float32)
            for dc in range(H // 16):
                off = dc * 16
                ai = ra_v[t, pl.ds(off, 16)]
                bi = rb_v[t, pl.ds(off, 16)]
                alo = plsc.bitcast(ai << 16, jnp.float32)
                ahi = plsc.bitcast(ai & -65536, jnp.float32)
                blo = plsc.bitcast(bi << 16, jnp.float32)
                bhi = plsc.bitcast(bi & -65536, jnp.float32)
                out_v[t, pl.ds(off, 16)] = g0 * alo + g1 * blo
                out_v[t, pl.ds(off + H, 16)] = g0 * ahi + g1 * bhi
            return carry

        lax.fori_loop(0, CT, tok_body, 0)
        odma[nb] = pltpu.async_copy(
            out_v, out_hbm.at[pl.ds(base + ci * CT, CT)], so[nb])
    for nb in range(2):
        if odma[nb] is not None:
            odma[nb].wait()


def kernel(x, Wg, bg, W1, b1, W2, b2):
    nt = B // TM
    f32 = jnp.float32
    lo = lambda i: (jnp.minimum(i, nt - 1), 0)
    hi = lambda i: (jnp.maximum(i - nt, 0), 0)
    const = lambda i: (0, 0)
    xpk, meta, wm, cnt, imp, ll, il, p0, p1, te = pl.pallas_call(
        _routing_body,
        grid=(2 * nt,),
        in_specs=[
            pl.BlockSpec((TM, D), lo),
            pl.BlockSpec((E, D), const),
            pl.BlockSpec((1, E), const),
        ],
        out_specs=[
            pl.BlockSpec((TM, H), lo),
            pl.BlockSpec((TM, 1), lo),
            pl.BlockSpec((TM, 1), lo),
            pl.BlockSpec((1, 16), const),
            pl.BlockSpec((1, E), const),
            pl.BlockSpec((1, 1), const),
            pl.BlockSpec((1, 1), const),
            pl.BlockSpec((TM, 1), hi),
            pl.BlockSpec((TM, 1), hi),
            pl.BlockSpec((NTE, 1), const),
        ],
        out_shape=[
            jax.ShapeDtypeStruct((B, H), jnp.int32),
            jax.ShapeDtypeStruct((B, 1), jnp.int32),
            jax.ShapeDtypeStruct((B, 1), jnp.int32),
            jax.ShapeDtypeStruct((1, 16), f32),
            jax.ShapeDtypeStruct((1, E), f32),
            jax.ShapeDtypeStruct((1, 1), f32),
            jax.ShapeDtypeStruct((1, 1), f32),
            jax.ShapeDtypeStruct((B, 1), jnp.int32),
            jax.ShapeDtypeStruct((B, 1), jnp.int32),
            jax.ShapeDtypeStruct((NTE, 1), jnp.int32),
        ],
        scratch_shapes=[pltpu.VMEM((B, 1), jnp.int32)],
    )(x, Wg, bg.reshape(1, E))

    metaf = meta.reshape(B)
    wmf = wm.reshape(B)
    cntf = cnt.reshape(16)
    te = te.reshape(NTE)
    xs = _sc_dispatch(xpk, p0.reshape(B), p1.reshape(B))

    w1b = W1.astype(jnp.bfloat16)
    w2b = W2.astype(jnp.bfloat16)
    grid_spec = pltpu.PrefetchScalarGridSpec(
        num_scalar_prefetch=1,
        grid=(NT2,),
        in_specs=[
            pl.BlockSpec((TM2, H), lambda i, te_r: (i, 0)),
            pl.BlockSpec((1, D, D), lambda i, te_r: (te_r[i], 0, 0)),
            pl.BlockSpec((1, 1, D), lambda i, te_r: (te_r[i], 0, 0)),
            pl.BlockSpec((1, D, D), lambda i, te_r: (te_r[i], 0, 0)),
            pl.BlockSpec((1, 1, D), lambda i, te_r: (te_r[i], 0, 0)),
        ],
        out_specs=pl.BlockSpec((TM2, H), lambda i, te_r: (i, 0)),
    )
    os_rows = pl.pallas_call(
        _gmlp_body,
        grid_spec=grid_spec,
        out_shape=jax.ShapeDtypeStruct((PMAX, H), jnp.int32),
    )(te[:NT2], xs, w1b, b1.reshape(E, 1, D), w2b, b2.reshape(E, 1, D))

    out = _sc_combine(os_rows, metaf, wmf, cntf)

    return out, ll.reshape(()), il.reshape(())


def _sc_mesh():
    return plsc.VectorSubcoreMesh(core_axis_name="c", subcore_axis_name="s",
                                  num_cores=2, num_subcores=16)


def _sc_dispatch(xpk, p0f, p1f):
    dispatch = functools.partial(
        pl.kernel,
        out_type=jax.ShapeDtypeStruct((PMAX, H), jnp.int32),
        mesh=_sc_mesh(),
        scratch_types=[
            pltpu.VMEM((TPW, H), jnp.int32),
            pltpu.VMEM((TPW,), jnp.int32),
            pltpu.VMEM((TPW,), jnp.int32),
            pltpu.SemaphoreType.DMA,
            pltpu.SemaphoreType.DMA,
            pltpu.SemaphoreType.DMA,
        ],
    )(_dispatch_body)
    return dispatch(xpk, p0f, p1f)


def _sc_combine(os_rows, metaf, wmf, cntf):
    f32 = jnp.float32
    combine = functools.partial(
        pl.kernel,
        out_type=jax.ShapeDtypeStruct((B, D), f32),
        mesh=_sc_mesh(),
        compiler_params=pltpu.CompilerParams(needs_layout_passes=False),
        scratch_types=[
            pltpu.VMEM((TPW,), jnp.int32),
            pltpu.VMEM((TPW,), jnp.int32),
            pltpu.VMEM((16,), f32),
            pltpu.VMEM((16,), jnp.int32),
            pltpu.VMEM((CT,), jnp.int32),
            pltpu.VMEM((CT,), jnp.int32),
            pltpu.VMEM((CT,), jnp.int32),
            pltpu.VMEM((CT,), jnp.int32),
            pltpu.VMEM((CT, H), jnp.int32),
            pltpu.VMEM((CT, H), jnp.int32),
            pltpu.VMEM((CT, H), jnp.int32),
            pltpu.VMEM((CT, H), jnp.int32),
            pltpu.VMEM((CT, D), f32),
            pltpu.VMEM((CT, D), f32),
            pltpu.SemaphoreType.DMA,
            pltpu.SemaphoreType.DMA,
            pltpu.SemaphoreType.DMA,
            pltpu.SemaphoreType.DMA,
            pltpu.SemaphoreType.DMA,
            pltpu.SemaphoreType.DMA,
            pltpu.SemaphoreType.DMA,
        ],
    )(_combine_body)
    return combine(os_rows, metaf, wmf, cntf)


# speedup vs baseline: 1.0837x; 1.0797x over previous
import functools

import jax
import jax.numpy as jnp
from jax import lax
from jax.experimental import pallas as pl
from jax.experimental.pallas import tpu as pltpu
from jax.experimental.pallas import tpu_sc as plsc

B = 4096
D = 1024
H = D // 2
E = 8
K = 2
TM = 512
TM2 = 256
LOG_TM2 = 8
PMAX = K * B + E * TM2
NT2 = PMAX // TM2
NTE = 48
NW = 32
TPW = B // NW
CT = 16
NCH = TPW // CT


def _rne_bf16_bits(v):
    u = jax.lax.bitcast_convert_type(v, jnp.uint32)
    return (u + jnp.uint32(0x7FFF) + ((u >> 16) & jnp.uint32(1))) >> 16


def _pack_bf16(v):
    r = _rne_bf16_bits(v)
    pk = r[:, :H] | (r[:, H:] << 16)
    return jax.lax.bitcast_convert_type(pk, jnp.int32)


def _unpack_bf16(pk):
    lo = jax.lax.bitcast_convert_type(pk << 16, jnp.float32)
    hi = jax.lax.bitcast_convert_type(pk & jnp.int32(-65536), jnp.float32)
    return jnp.concatenate([lo, hi], axis=1)


def _routing_body(x_ref, wg_ref, bg_ref, xpk_ref, meta_ref, wm_ref,
                  cnt_ref, imp_ref, ll_ref, il_ref, p0_ref, p1_ref, te_ref,
                  mscr):
    i = pl.program_id(0)
    nt = B // TM

    @pl.when(i == 0)
    def _():
        cnt_ref[...] = jnp.zeros((1, 16), jnp.float32)
        imp_ref[...] = jnp.zeros((1, E), jnp.float32)

    @pl.when(i < nt)
    def _():
        cb = cnt_ref[...][:, :E]
        x = x_ref[...]
        xpk_ref[...] = _pack_bf16(x)
        s = jax.lax.dot_general(
            x, wg_ref[...], (((1,), (1,)), ((), ())),
            preferred_element_type=jnp.float32) + bg_ref[...]
        ids = jax.lax.broadcasted_iota(jnp.int32, (TM, E), 1)
        m1 = jnp.max(s, axis=1, keepdims=True)
        a1v = jnp.min(jnp.where(s == m1, ids, E), axis=1, keepdims=True)
        s2 = jnp.where(ids == a1v, -jnp.inf, s)
        m2 = jnp.max(s2, axis=1, keepdims=True)
        a2v = jnp.min(jnp.where(s2 == m2, ids, E), axis=1, keepdims=True)
        e21 = jnp.exp(m2 - m1)
        w0 = 1.0 / (1.0 + e21)
        w1 = e21 / (1.0 + e21)
        wbits = (_rne_bf16_bits(w0) | (_rne_bf16_bits(w1) << 16))
        wm_ref[...] = jax.lax.bitcast_convert_type(wbits, jnp.int32)
        is1 = (ids == a1v).astype(jnp.float32)
        is2 = (ids == a2v).astype(jnp.float32)
        m = is1 + is2
        c = m
        sh = 1
        while sh < TM:
            c = c + jnp.concatenate(
                [jnp.zeros((sh, E), jnp.float32), c[:TM - sh]], axis=0)
            sh *= 2
        cexc = c - m
        r0 = jnp.sum(is1 * (cexc + cb), axis=1,
                     keepdims=True).astype(jnp.int32)
        r1 = jnp.sum(is2 * (cexc + is1 + cb), axis=1,
                     keepdims=True).astype(jnp.int32)
        mv = a1v | (r0 << 3) | (a2v << 16) | (r1 << 19)
        meta_ref[...] = mv
        mscr[pl.ds(i * TM, TM), :] = mv
        cpart = jnp.sum(m, axis=0, keepdims=True)
        cnt_ref[...] += jnp.concatenate(
            [cpart, jnp.zeros((1, 16 - E), jnp.float32)], axis=1)
        ex = jnp.exp(s - m1)
        sm = ex / jnp.sum(ex, axis=1, keepdims=True)
        imp_ref[...] += jnp.sum(sm, axis=0, keepdims=True)

    @pl.when(i == nt - 1)
    def _():
        cfin = cnt_ref[...][:, :E]
        cm = jnp.sum(cfin) / E
        cvar = jnp.sum((cfin - cm) ** 2) / (E - 1)
        ll_ref[...] = cvar.reshape(1, 1) / (E * (B / E))
        im = imp_ref[...]
        imm = jnp.sum(im) / E
        ivar = jnp.sum((im - imm) ** 2) / (E - 1)
        il_ref[...] = ivar.reshape(1, 1) / (imm + 1e-8)

    @pl.when(i >= nt)
    def _():
        c = cnt_ref[...][:, :E]
        pc = jnp.ceil(c / TM2) * TM2
        lt = (jax.lax.broadcasted_iota(jnp.int32, (E, E), 0) <
              jax.lax.broadcasted_iota(jnp.int32, (E, E), 1)
              ).astype(jnp.float32)
        offs = jax.lax.dot_general(pc, lt, (((1,), (0,)), ((), ())),
                                   preferred_element_type=jnp.float32)
        mv = mscr[pl.ds((i - nt) * TM, TM), :]
        a0 = mv & 7
        r0 = (mv >> 3) & 0x1FFF
        a1 = (mv >> 16) & 7
        r1 = lax.shift_right_logical(mv, 19)
        iot = jax.lax.broadcasted_iota(jnp.int32, (TM, E), 1)
        for a, r, p_ref in ((a0, r0, p0_ref), (a1, r1, p1_ref)):
            oh = (a == iot).astype(jnp.float32)
            osel = jnp.sum(oh * offs, axis=1, keepdims=True)
            p_ref[...] = osel.astype(jnp.int32) + r

        @pl.when(i == nt)
        def _():
            ends = offs + pc
            starts = (jax.lax.broadcasted_iota(jnp.int32, (NTE, 1), 0)
                      * TM2).astype(jnp.float32)
            cmp = (starts >= ends).astype(jnp.int32)
            te_ref[...] = jnp.minimum(jnp.sum(cmp, axis=1, keepdims=True),
                                      E - 1)


def _pos_body(meta_ref, cnt_ref, p0_ref, p1_ref, te_ref):
    m = meta_ref[...]
    a0 = m & 7
    r0 = (m >> 3) & 0x1FFF
    a1 = (m >> 16) & 7
    r1 = lax.shift_right_logical(m, 19)
    c = cnt_ref[...][:, :E]
    pc = jnp.ceil(c / TM2) * TM2
    lt = (jax.lax.broadcasted_iota(jnp.int32, (E, E), 0) <
          jax.lax.broadcasted_iota(jnp.int32, (E, E), 1)).astype(jnp.float32)
    offs = jax.lax.dot_general(pc, lt, (((1,), (0,)), ((), ())),
                               preferred_element_type=jnp.float32)
    iot = jax.lax.broadcasted_iota(jnp.int32, (TM, E), 1)
    for a, r, p_ref in ((a0, r0, p0_ref), (a1, r1, p1_ref)):
        oh = (a == iot).astype(jnp.float32)
        osel = jnp.sum(oh * offs, axis=1, keepdims=True)
        p_ref[...] = osel.astype(jnp.int32) + r

    @pl.when(pl.program_id(0) == 0)
    def _():
        ends = offs + pc
        starts = (jax.lax.broadcasted_iota(jnp.int32, (NTE, 1), 0)
                  * TM2).astype(jnp.float32)
        cmp = (starts >= ends).astype(jnp.int32)
        te_ref[...] = jnp.minimum(jnp.sum(cmp, axis=1, keepdims=True), E - 1)


def _gmlp_body(te_ref, xs_ref, w1_ref, b1_ref, w2_ref, b2_ref, o_ref):
    xb = _unpack_bf16(xs_ref[...]).astype(jnp.bfloat16)
    h = jax.lax.dot_general(
        xb, w1_ref[0], (((1,), (1,)), ((), ())),
        preferred_element_type=jnp.float32) + b1_ref[0]
    hb = jnp.maximum(h, 0.0).astype(jnp.bfloat16)
    o_ref[...] = jax.lax.dot_general(
        hb, w2_ref[0], (((1,), (1,)), ((), ())),
        preferred_element_type=jnp.float32) + b2_ref[0]


def _offsets(cnt_v, offs_v):
    ci = cnt_v[...].astype(jnp.int32)
    pci = ((ci + (TM2 - 1)) >> LOG_TM2) << LOG_TM2
    incl = plsc.cumsum(pci)
    excl = incl - pci
    offs_v[...] = excl
    return excl, pci


def _slots(m, offs_v):
    a0 = m & 7
    r0 = (m >> 3) & 0x1FFF
    a1 = (m >> 16) & 7
    r1 = lax.shift_right_logical(m, 19)
    s0 = r0 + plsc.load_gather(offs_v, [a0])
    s1 = r1 + plsc.load_gather(offs_v, [a1])
    return s0, s1


def _dispatch_body(xpk_hbm, p0_hbm, p1_hbm, xs_hbm,
                   rows_v, i0_v, i1_v, seml, sem0, sem1):
    wid = lax.axis_index("s") * 2 + lax.axis_index("c")
    base = wid * TPW
    l0 = pltpu.async_copy(p0_hbm.at[pl.ds(base, TPW)], i0_v, seml)
    l1 = pltpu.async_copy(p1_hbm.at[pl.ds(base, TPW)], i1_v, seml)
    l2 = pltpu.async_copy(xpk_hbm.at[pl.ds(base, TPW)], rows_v, seml)
    l0.wait()
    l1.wait()
    l2.wait()
    c0 = pltpu.async_copy(rows_v, xs_hbm.at[i0_v], sem0)
    c1 = pltpu.async_copy(rows_v, xs_hbm.at[i1_v], sem1)
    c0.wait()
    c1.wait()


def _combine_body(os_hbm, meta_hbm, wm_hbm, cnt_hbm, out_hbm,
                  m_v, wm_v, cnt_v, offs_v,
                  ia0_v, ia1_v, ib0_v, ib1_v,
                  ra0_v, ra1_v, rb0_v, rb1_v, oc0_v, oc1_v,
                  seml, sema0, sema1, semb0, semb1, semo0, semo1):
    wid = lax.axis_index("s") * 2 + lax.axis_index("c")
    base = wid * TPW
    l0 = pltpu.async_copy(meta_hbm.at[pl.ds(base, TPW)], m_v, seml)
    l1 = pltpu.async_copy(wm_hbm.at[pl.ds(base, TPW)], wm_v, seml)
    pltpu.sync_copy(cnt_hbm, cnt_v)
    _offsets(cnt_v, offs_v)
    l0.wait()
    l1.wait()
    ia = (ia0_v, ia1_v)
    ib = (ib0_v, ib1_v)
    ra = (ra0_v, ra1_v)
    rb = (rb0_v, rb1_v)
    oc = (oc0_v, oc1_v)
    sa = (sema0, sema1)
    sb = (semb0, semb1)
    so = (semo0, semo1)
    gat = [None, None]
    odma = [None, None]

    def start(ci):
        nb = ci % 2
        m = m_v[pl.ds(ci * CT, CT)]
        s0, s1 = _slots(m, offs_v)
        ia[nb][...] = s0
        ib[nb][...] = s1
        gat[nb] = (pltpu.async_copy(os_hbm.at[ia[nb]], ra[nb], sa[nb]),
                   pltpu.async_copy(os_hbm.at[ib[nb]], rb[nb], sb[nb]))

    start(0)
    for ci in range(NCH):
        nb = ci % 2
        if ci + 1 < NCH:
            start(ci + 1)
        gat[nb][0].wait()
        gat[nb][1].wait()
        if odma[nb] is not None:
            odma[nb].wait()
        ra_v = ra[nb]
        rb_v = rb[nb]
        out_v = oc[nb]

        def tok_body(t, carry):
            wm = plsc.load_gather(wm_v, [jnp.full((16,), ci * CT + t,
                                                  jnp.int32)])
            g0 = plsc.bitcast(wm << 16, jnp.float32)
            g1 = plsc.bitcast(wm & -65536, jnp.float32)
            for dc in range(D // 16):
                off = dc * 16
                out_v[t, pl.ds(off, 16)] = (g0 * ra_v[t, pl.ds(off, 16)] +
                                            g1 * rb_v[t, pl.ds(off, 16)])
            return carry

        lax.fori_loop(0, CT, tok_body, 0)
        odma[nb] = pltpu.async_copy(
            out_v, out_hbm.at[pl.ds(base + ci * CT, CT)], so[nb])
    for nb in range(2):
        if odma[nb] is not None:
            odma[nb].wait()


def kernel(x, Wg, bg, W1, b1, W2, b2):
    nt = B // TM
    f32 = jnp.float32
    lo = lambda i: (jnp.minimum(i, nt - 1), 0)
    hi = lambda i: (jnp.maximum(i - nt, 0), 0)
    const = lambda i: (0, 0)
    xpk, meta, wm, cnt, imp, ll, il, p0, p1, te = pl.pallas_call(
        _routing_body,
        grid=(2 * nt,),
        in_specs=[
            pl.BlockSpec((TM, D), lo),
            pl.BlockSpec((E, D), const),
            pl.BlockSpec((1, E), const),
        ],
        out_specs=[
            pl.BlockSpec((TM, H), lo),
            pl.BlockSpec((TM, 1), lo),
            pl.BlockSpec((TM, 1), lo),
            pl.BlockSpec((1, 16), const),
            pl.BlockSpec((1, E), const),
            pl.BlockSpec((1, 1), const),
            pl.BlockSpec((1, 1), const),
            pl.BlockSpec((TM, 1), hi),
            pl.BlockSpec((TM, 1), hi),
            pl.BlockSpec((NTE, 1), const),
        ],
        out_shape=[
            jax.ShapeDtypeStruct((B, H), jnp.int32),
            jax.ShapeDtypeStruct((B, 1), jnp.int32),
            jax.ShapeDtypeStruct((B, 1), jnp.int32),
            jax.ShapeDtypeStruct((1, 16), f32),
            jax.ShapeDtypeStruct((1, E), f32),
            jax.ShapeDtypeStruct((1, 1), f32),
            jax.ShapeDtypeStruct((1, 1), f32),
            jax.ShapeDtypeStruct((B, 1), jnp.int32),
            jax.ShapeDtypeStruct((B, 1), jnp.int32),
            jax.ShapeDtypeStruct((NTE, 1), jnp.int32),
        ],
        scratch_shapes=[pltpu.VMEM((B, 1), jnp.int32)],
    )(x, Wg, bg.reshape(1, E))

    metaf = meta.reshape(B)
    wmf = wm.reshape(B)
    cntf = cnt.reshape(16)
    te = te.reshape(NTE)
    xs = _sc_dispatch(xpk, p0.reshape(B), p1.reshape(B))

    w1b = W1.astype(jnp.bfloat16)
    w2b = W2.astype(jnp.bfloat16)
    grid_spec = pltpu.PrefetchScalarGridSpec(
        num_scalar_prefetch=1,
        grid=(NT2,),
        in_specs=[
            pl.BlockSpec((TM2, H), lambda i, te_r: (i, 0)),
            pl.BlockSpec((1, D, D), lambda i, te_r: (te_r[i], 0, 0)),
            pl.BlockSpec((1, 1, D), lambda i, te_r: (te_r[i], 0, 0)),
            pl.BlockSpec((1, D, D), lambda i, te_r: (te_r[i], 0, 0)),
            pl.BlockSpec((1, 1, D), lambda i, te_r: (te_r[i], 0, 0)),
        ],
        out_specs=pl.BlockSpec((TM2, D), lambda i, te_r: (i, 0)),
    )
    os_rows = pl.pallas_call(
        _gmlp_body,
        grid_spec=grid_spec,
        out_shape=jax.ShapeDtypeStruct((PMAX, D), jnp.float32),
    )(te[:NT2], xs, w1b, b1.reshape(E, 1, D), w2b, b2.reshape(E, 1, D))

    out = _sc_combine(os_rows, metaf, wmf, cntf)

    return out, ll.reshape(()), il.reshape(())


def _sc_mesh():
    return plsc.VectorSubcoreMesh(core_axis_name="c", subcore_axis_name="s",
                                  num_cores=2, num_subcores=16)


def _sc_dispatch(xpk, p0f, p1f):
    dispatch = functools.partial(
        pl.kernel,
        out_type=jax.ShapeDtypeStruct((PMAX, H), jnp.int32),
        mesh=_sc_mesh(),
        scratch_types=[
            pltpu.VMEM((TPW, H), jnp.int32),
            pltpu.VMEM((TPW,), jnp.int32),
            pltpu.VMEM((TPW,), jnp.int32),
            pltpu.SemaphoreType.DMA,
            pltpu.SemaphoreType.DMA,
            pltpu.SemaphoreType.DMA,
        ],
    )(_dispatch_body)
    return dispatch(xpk, p0f, p1f)


def _sc_combine(os_rows, metaf, wmf, cntf):
    f32 = jnp.float32
    combine = functools.partial(
        pl.kernel,
        out_type=jax.ShapeDtypeStruct((B, D), f32),
        mesh=_sc_mesh(),
        compiler_params=pltpu.CompilerParams(needs_layout_passes=False),
        scratch_types=[
            pltpu.VMEM((TPW,), jnp.int32),
            pltpu.VMEM((TPW,), jnp.int32),
            pltpu.VMEM((16,), f32),
            pltpu.VMEM((16,), jnp.int32),
            pltpu.VMEM((CT,), jnp.int32),
            pltpu.VMEM((CT,), jnp.int32),
            pltpu.VMEM((CT,), jnp.int32),
            pltpu.VMEM((CT,), jnp.int32),
            pltpu.VMEM((CT, D), f32),
            pltpu.VMEM((CT, D), f32),
            pltpu.VMEM((CT, D), f32),
            pltpu.VMEM((CT, D), f32),
            pltpu.VMEM((CT, D), f32),
            pltpu.VMEM((CT, D), f32),
            pltpu.SemaphoreType.DMA,
            pltpu.SemaphoreType.DMA,
            pltpu.SemaphoreType.DMA,
            pltpu.SemaphoreType.DMA,
            pltpu.SemaphoreType.DMA,
            pltpu.SemaphoreType.DMA,
            pltpu.SemaphoreType.DMA,
        ],
    )(_combine_body)
    return combine(os_rows, metaf, wmf, cntf)


# speedup vs baseline: 1.0837x; 1.0000x over previous
import functools

import jax
import jax.numpy as jnp
from jax import lax
from jax.experimental import pallas as pl
from jax.experimental.pallas import tpu as pltpu
from jax.experimental.pallas import tpu_sc as plsc

B = 4096
D = 1024
H = D // 2
E = 8
K = 2
TM = 512
TM2 = 256
LOG_TM2 = 8
PMAX = K * B + E * TM2
NT2 = PMAX // TM2
NTE = 48
NW = 32
TPW = B // NW
CT = 16
NCH = TPW // CT


def _rne_bf16_bits(v):
    u = jax.lax.bitcast_convert_type(v, jnp.uint32)
    return (u + jnp.uint32(0x7FFF) + ((u >> 16) & jnp.uint32(1))) >> 16


def _pack_bf16(v):
    r = _rne_bf16_bits(v)
    pk = r[:, :H] | (r[:, H:] << 16)
    return jax.lax.bitcast_convert_type(pk, jnp.int32)


def _unpack_bf16(pk):
    lo = jax.lax.bitcast_convert_type(pk << 16, jnp.float32)
    hi = jax.lax.bitcast_convert_type(pk & jnp.int32(-65536), jnp.float32)
    return jnp.concatenate([lo, hi], axis=1)


def _routing_body(x_ref, wg_ref, bg_ref, xpk_ref, meta_ref, wm_ref,
                  cnt_ref, imp_ref, ll_ref, il_ref, p0_ref, p1_ref, te_ref,
                  mscr):
    i = pl.program_id(0)
    nt = B // TM

    @pl.when(i == 0)
    def _():
        cnt_ref[...] = jnp.zeros((1, 16), jnp.float32)
        imp_ref[...] = jnp.zeros((1, E), jnp.float32)

    @pl.when(i < nt)
    def _():
        cb = cnt_ref[...][:, :E]
        x = x_ref[...]
        xpk_ref[...] = _pack_bf16(x)
        s = jax.lax.dot_general(
            x, wg_ref[...], (((1,), (1,)), ((), ())),
            preferred_element_type=jnp.float32) + bg_ref[...]
        ids = jax.lax.broadcasted_iota(jnp.int32, (TM, E), 1)
        m1 = jnp.max(s, axis=1, keepdims=True)
        a1v = jnp.min(jnp.where(s == m1, ids, E), axis=1, keepdims=True)
        s2 = jnp.where(ids == a1v, -jnp.inf, s)
        m2 = jnp.max(s2, axis=1, keepdims=True)
        a2v = jnp.min(jnp.where(s2 == m2, ids, E), axis=1, keepdims=True)
        e21 = jnp.exp(m2 - m1)
        w0 = 1.0 / (1.0 + e21)
        w1 = e21 / (1.0 + e21)
        wbits = (_rne_bf16_bits(w0) | (_rne_bf16_bits(w1) << 16))
        wm_ref[...] = jax.lax.bitcast_convert_type(wbits, jnp.int32)
        is1 = (ids == a1v).astype(jnp.float32)
        is2 = (ids == a2v).astype(jnp.float32)
        m = is1 + is2
        c = m
        sh = 1
        while sh < TM:
            c = c + jnp.concatenate(
                [jnp.zeros((sh, E), jnp.float32), c[:TM - sh]], axis=0)
            sh *= 2
        cexc = c - m
        r0 = jnp.sum(is1 * (cexc + cb), axis=1,
                     keepdims=True).astype(jnp.int32)
        r1 = jnp.sum(is2 * (cexc + is1 + cb), axis=1,
                     keepdims=True).astype(jnp.int32)
        mv = a1v | (r0 << 3) | (a2v << 16) | (r1 << 19)
        meta_ref[...] = mv
        mscr[pl.ds(i * TM, TM), :] = mv
        cpart = jnp.sum(m, axis=0, keepdims=True)
        cnt_ref[...] += jnp.concatenate(
            [cpart, jnp.zeros((1, 16 - E), jnp.float32)], axis=1)
        ex = jnp.exp(s - m1)
        sm = ex / jnp.sum(ex, axis=1, keepdims=True)
        imp_ref[...] += jnp.sum(sm, axis=0, keepdims=True)

    @pl.when(i == nt - 1)
    def _():
        cfin = cnt_ref[...][:, :E]
        cm = jnp.sum(cfin) / E
        cvar = jnp.sum((cfin - cm) ** 2) / (E - 1)
        ll_ref[...] = cvar.reshape(1, 1) / (E * (B / E))
        im = imp_ref[...]
        imm = jnp.sum(im) / E
        ivar = jnp.sum((im - imm) ** 2) / (E - 1)
        il_ref[...] = ivar.reshape(1, 1) / (imm + 1e-8)

    @pl.when(i >= nt)
    def _():
        c = cnt_ref[...][:, :E]
        pc = jnp.ceil(c / TM2) * TM2
        lt = (jax.lax.broadcasted_iota(jnp.int32, (E, E), 0) <
              jax.lax.broadcasted_iota(jnp.int32, (E, E), 1)
              ).astype(jnp.float32)
        offs = jax.lax.dot_general(pc, lt, (((1,), (0,)), ((), ())),
                                   preferred_element_type=jnp.float32)
        mv = mscr[pl.ds((i - nt) * TM, TM), :]
        a0 = mv & 7
        r0 = (mv >> 3) & 0x1FFF
        a1 = (mv >> 16) & 7
        r1 = lax.shift_right_logical(mv, 19)
        iot = jax.lax.broadcasted_iota(jnp.int32, (TM, E), 1)
        for a, r, p_ref in ((a0, r0, p0_ref), (a1, r1, p1_ref)):
            oh = (a == iot).astype(jnp.float32)
            osel = jnp.sum(oh * offs, axis=1, keepdims=True)
            p_ref[...] = osel.astype(jnp.int32) + r

        @pl.when(i == nt)
        def _():
            ends = offs + pc
            starts = (jax.lax.broadcasted_iota(jnp.int32, (NTE, 1), 0)
                      * TM2).astype(jnp.float32)
            cmp = (starts >= ends).astype(jnp.int32)
            te_ref[...] = jnp.minimum(jnp.sum(cmp, axis=1, keepdims=True),
                                      E - 1)


def _gmlp_body(te_ref, xs_ref, w1_ref, b1_ref, w2_ref, b2_ref, o_ref):
    xb = _unpack_bf16(xs_ref[...]).astype(jnp.bfloat16)
    h = jax.lax.dot_general(
        xb, w1_ref[0], (((1,), (1,)), ((), ())),
        preferred_element_type=jnp.float32) + b1_ref[0]
    hb = jnp.maximum(h, 0.0).astype(jnp.bfloat16)
    o_ref[...] = jax.lax.dot_general(
        hb, w2_ref[0], (((1,), (1,)), ((), ())),
        preferred_element_type=jnp.float32) + b2_ref[0]


def _offsets(cnt_v, offs_v):
    ci = cnt_v[...].astype(jnp.int32)
    pci = ((ci + (TM2 - 1)) >> LOG_TM2) << LOG_TM2
    incl = plsc.cumsum(pci)
    excl = incl - pci
    offs_v[...] = excl
    return excl, pci


def _slots(m, offs_v):
    a0 = m & 7
    r0 = (m >> 3) & 0x1FFF
    a1 = (m >> 16) & 7
    r1 = lax.shift_right_logical(m, 19)
    s0 = r0 + plsc.load_gather(offs_v, [a0])
    s1 = r1 + plsc.load_gather(offs_v, [a1])
    return s0, s1


def _dispatch_body(xpk_hbm, p0_hbm, p1_hbm, xs_hbm,
                   rows_v, i0_v, i1_v, seml, sem0, sem1):
    wid = lax.axis_index("s") * 2 + lax.axis_index("c")
    base = wid * TPW
    l0 = pltpu.async_copy(p0_hbm.at[pl.ds(base, TPW)], i0_v, seml)
    l1 = pltpu.async_copy(p1_hbm.at[pl.ds(base, TPW)], i1_v, seml)
    l2 = pltpu.async_copy(xpk_hbm.at[pl.ds(base, TPW)], rows_v, seml)
    l0.wait()
    l1.wait()
    l2.wait()
    c0 = pltpu.async_copy(rows_v, xs_hbm.at[i0_v], sem0)
    c1 = pltpu.async_copy(rows_v, xs_hbm.at[i1_v], sem1)
    c0.wait()
    c1.wait()


def _combine_body(os_hbm, meta_hbm, wm_hbm, cnt_hbm, out_hbm,
                  m_v, wm_v, cnt_v, offs_v,
                  ia0_v, ia1_v, ib0_v, ib1_v,
                  ra0_v, ra1_v, rb0_v, rb1_v, oc0_v, oc1_v,
                  seml, sema0, sema1, semb0, semb1, semo0, semo1):
    wid = lax.axis_index("s") * 2 + lax.axis_index("c")
    base = wid * TPW
    l0 = pltpu.async_copy(meta_hbm.at[pl.ds(base, TPW)], m_v, seml)
    l1 = pltpu.async_copy(wm_hbm.at[pl.ds(base, TPW)], wm_v, seml)
    pltpu.sync_copy(cnt_hbm, cnt_v)
    _offsets(cnt_v, offs_v)
    l0.wait()
    l1.wait()
    ia = (ia0_v, ia1_v)
    ib = (ib0_v, ib1_v)
    ra = (ra0_v, ra1_v)
    rb = (rb0_v, rb1_v)
    oc = (oc0_v, oc1_v)
    sa = (sema0, sema1)
    sb = (semb0, semb1)
    so = (semo0, semo1)
    gat = [None, None]
    odma = [None, None]

    def start(ci):
        nb = ci % 2
        m = m_v[pl.ds(ci * CT, CT)]
        s0, s1 = _slots(m, offs_v)
        ia[nb][...] = s0
        ib[nb][...] = s1
        gat[nb] = (pltpu.async_copy(os_hbm.at[ia[nb]], ra[nb], sa[nb]),
                   pltpu.async_copy(os_hbm.at[ib[nb]], rb[nb], sb[nb]))

    start(0)
    for ci in range(NCH):
        nb = ci % 2
        if ci + 1 < NCH:
            start(ci + 1)
        gat[nb][0].wait()
        gat[nb][1].wait()
        if odma[nb] is not None:
            odma[nb].wait()
        ra_v = ra[nb]
        rb_v = rb[nb]
        out_v = oc[nb]

        def tok_body(t, carry):
            wm = plsc.load_gather(wm_v, [jnp.full((16,), ci * CT + t,
                                                  jnp.int32)])
            g0 = plsc.bitcast(wm << 16, jnp.float32)
            g1 = plsc.bitcast(wm & -65536, jnp.float32)
            for dc in range(D // 16):
                off = dc * 16
                out_v[t, pl.ds(off, 16)] = (g0 * ra_v[t, pl.ds(off, 16)] +
                                            g1 * rb_v[t, pl.ds(off, 16)])
            return carry

        lax.fori_loop(0, CT, tok_body, 0)
        odma[nb] = pltpu.async_copy(
            out_v, out_hbm.at[pl.ds(base + ci * CT, CT)], so[nb])
    for nb in range(2):
        if odma[nb] is not None:
            odma[nb].wait()


def kernel(x, Wg, bg, W1, b1, W2, b2):
    nt = B // TM
    f32 = jnp.float32
    lo = lambda i: (jnp.minimum(i, nt - 1), 0)
    hi = lambda i: (jnp.maximum(i - nt, 0), 0)
    const = lambda i: (0, 0)
    xpk, meta, wm, cnt, imp, ll, il, p0, p1, te = pl.pallas_call(
        _routing_body,
        grid=(2 * nt,),
        in_specs=[
            pl.BlockSpec((TM, D), lo),
            pl.BlockSpec((E, D), const),
            pl.BlockSpec((1, E), const),
        ],
        out_specs=[
            pl.BlockSpec((TM, H), lo),
            pl.BlockSpec((TM, 1), lo),
            pl.BlockSpec((TM, 1), lo),
            pl.BlockSpec((1, 16), const),
            pl.BlockSpec((1, E), const),
            pl.BlockSpec((1, 1), const),
            pl.BlockSpec((1, 1), const),
            pl.BlockSpec((TM, 1), hi),
            pl.BlockSpec((TM, 1), hi),
            pl.BlockSpec((NTE, 1), const),
        ],
        out_shape=[
            jax.ShapeDtypeStruct((B, H), jnp.int32),
            jax.ShapeDtypeStruct((B, 1), jnp.int32),
            jax.ShapeDtypeStruct((B, 1), jnp.int32),
            jax.ShapeDtypeStruct((1, 16), f32),
            jax.ShapeDtypeStruct((1, E), f32),
            jax.ShapeDtypeStruct((1, 1), f32),
            jax.ShapeDtypeStruct((1, 1), f32),
            jax.ShapeDtypeStruct((B, 1), jnp.int32),
            jax.ShapeDtypeStruct((B, 1), jnp.int32),
            jax.ShapeDtypeStruct((NTE, 1), jnp.int32),
        ],
        scratch_shapes=[pltpu.VMEM((B, 1), jnp.int32)],
    )(x, Wg, bg.reshape(1, E))

    metaf = meta.reshape(B)
    wmf = wm.reshape(B)
    cntf = cnt.reshape(16)
    te = te.reshape(NTE)
    xs = _sc_dispatch(xpk, p0.reshape(B), p1.reshape(B))

    w1b = W1.astype(jnp.bfloat16)
    w2b = W2.astype(jnp.bfloat16)
    grid_spec = pltpu.PrefetchScalarGridSpec(
        num_scalar_prefetch=1,
        grid=(NT2,),
        in_specs=[
            pl.BlockSpec((TM2, H), lambda i, te_r: (i, 0)),
            pl.BlockSpec((1, D, D), lambda i, te_r: (te_r[i], 0, 0)),
            pl.BlockSpec((1, 1, D), lambda i, te_r: (te_r[i], 0, 0)),
            pl.BlockSpec((1, D, D), lambda i, te_r: (te_r[i], 0, 0)),
            pl.BlockSpec((1, 1, D), lambda i, te_r: (te_r[i], 0, 0)),
        ],
        out_specs=pl.BlockSpec((TM2, D), lambda i, te_r: (i, 0)),
    )
    os_rows = pl.pallas_call(
        _gmlp_body,
        grid_spec=grid_spec,
        out_shape=jax.ShapeDtypeStruct((PMAX, D), jnp.float32),
    )(te[:NT2], xs, w1b, b1.reshape(E, 1, D), w2b, b2.reshape(E, 1, D))

    out = _sc_combine(os_rows, metaf, wmf, cntf)

    return out, ll.reshape(()), il.reshape(())


def _sc_mesh():
    return plsc.VectorSubcoreMesh(core_axis_name="c", subcore_axis_name="s",
                                  num_cores=2, num_subcores=16)


def _sc_dispatch(xpk, p0f, p1f):
    dispatch = functools.partial(
        pl.kernel,
        out_type=jax.ShapeDtypeStruct((PMAX, H), jnp.int32),
        mesh=_sc_mesh(),
        scratch_types=[
            pltpu.VMEM((TPW, H), jnp.int32),
            pltpu.VMEM((TPW,), jnp.int32),
            pltpu.VMEM((TPW,), jnp.int32),
            pltpu.SemaphoreType.DMA,
            pltpu.SemaphoreType.DMA,
            pltpu.SemaphoreType.DMA,
        ],
    )(_dispatch_body)
    return dispatch(xpk, p0f, p1f)


def _sc_combine(os_rows, metaf, wmf, cntf):
    f32 = jnp.float32
    combine = functools.partial(
        pl.kernel,
        out_type=jax.ShapeDtypeStruct((B, D), f32),
        mesh=_sc_mesh(),
        compiler_params=pltpu.CompilerParams(needs_layout_passes=False),
        scratch_types=[
            pltpu.VMEM((TPW,), jnp.int32),
            pltpu.VMEM((TPW,), jnp.int32),
            pltpu.VMEM((16,), f32),
            pltpu.VMEM((16,), jnp.int32),
            pltpu.VMEM((CT,), jnp.int32),
            pltpu.VMEM((CT,), jnp.int32),
            pltpu.VMEM((CT,), jnp.int32),
            pltpu.VMEM((CT,), jnp.int32),
            pltpu.VMEM((CT, D), f32),
            pltpu.VMEM((CT, D), f32),
            pltpu.VMEM((CT, D), f32),
            pltpu.VMEM((CT, D), f32),
            pltpu.VMEM((CT, D), f32),
            pltpu.VMEM((CT, D), f32),
            pltpu.SemaphoreType.DMA,
            pltpu.SemaphoreType.DMA,
            pltpu.SemaphoreType.DMA,
            pltpu.SemaphoreType.DMA,
            pltpu.SemaphoreType.DMA,
            pltpu.SemaphoreType.DMA,
            pltpu.SemaphoreType.DMA,
        ],
    )(_combine_body)
    return combine(os_rows, metaf, wmf, cntf)
